# Initial kernel scaffold; baseline (speedup 1.0000x reference)
import jax
import jax.numpy as jnp
from jax.experimental import pallas as pl


def kernel(x, lengths, w1, b1, w2, b2, pw, pb):
    raise NotImplementedError("write your pallas kernel here")


# speedup vs baseline: 2.5585x; 2.5585x over previous
import jax
import jax.numpy as jnp
from jax.experimental import pallas as pl
from jax.experimental.pallas import tpu as pltpu

NF = 32
NB = 128
NT = 1024
NH = 16
NK = 5
ND = 4
G = 8
NGRP = NF // G
TC = 64
NC = NT // TC
E = TC + 4
CPS = (NT // ND) // TC
EB = E * NB
TB = TC * NB
ROWS = G * NH


def _dot(a, b):
    return jax.lax.dot_general(a, b, (((1,), (0,)), ((), ())),
                               preferred_element_type=jnp.float32)


def _kernel_body(x_ref, w1_ref, b1_ref, wp01_ref, wp23_ref, w4_ref, b2_ref,
                 pw_ref, pb_ref, o_ref):
    xg = x_ref[0]
    w1 = w1_ref[0]
    b1v = b1_ref[0]
    wp01 = wp01_ref[0]
    wp23 = wp23_ref[0]
    w4 = w4_ref[0]
    b2v = b2_ref[0]
    segs = [None] * ND
    for c in range(NC):
        base = c * TC * NB
        xim = jnp.concatenate(
            [xg[:, base + k * NB: base + k * NB + EB] for k in range(NK)],
            axis=0).astype(jnp.bfloat16)
        h1 = jnp.maximum(_dot(w1, xim) + b1v, 0.0)
        if c == 0:
            iot = jax.lax.broadcasted_iota(jnp.int32, (1, EB), 1)
            h1 = jnp.where(iot >= 2 * NB, h1, 0.0)
        if c == NC - 1:
            iot = jax.lax.broadcasted_iota(jnp.int32, (1, EB), 1)
            h1 = jnp.where(iot < (E - 2) * NB, h1, 0.0)
        h1b = h1.astype(jnp.bfloat16)
        w2c = (E - 1) * NB
        h2in = jnp.concatenate([h1b[:, :w2c], h1b[:, NB:NB + w2c]], axis=0)
        s = _dot(wp01, h2in[:, :TB])
        s = s + _dot(wp23, h2in[:, 2 * NB: 2 * NB + TB])
        s = s + _dot(w4, h1b[:, 4 * NB: 4 * NB + TB])
        h2 = jnp.maximum(s + b2v, 0.0)
        r = h2
        w = TB
        while w > NB:
            w //= 2
            r = r[:, :w] + r[:, w: 2 * w]
        si = c // CPS
        segs[si] = r if segs[si] is None else segs[si] + r
    pmat = jnp.concatenate(segs, axis=0)
    o_ref[0] = _dot(pw_ref[0], pmat) + pb_ref[0]


def kernel(x, lengths, w1, b1, w2, b2, pw, pb):
    del lengths
    xt = jnp.transpose(x, (2, 1, 0))
    xp = jnp.pad(xt, ((0, 0), (4, 4), (0, 0)))
    x2 = xp.reshape(NGRP, G, (NT + 8) * NB)

    eye = jnp.eye(G, dtype=jnp.float32)
    w1r = w1[:, :, 0, :].reshape(NGRP, G, NH, NK)
    w1bd = jnp.einsum('gfck,fe->gfcke', w1r, eye)
    w1bd = w1bd.reshape(NGRP, ROWS, NK * G).astype(jnp.bfloat16)
    b1g = b1.reshape(NGRP, ROWS, 1)
    w2r = w2.reshape(NGRP, G, NH, NH, NK)
    w2bd = jnp.einsum('gfoik,fe->gkfoei', w2r, eye).reshape(NGRP, NK, ROWS, ROWS)
    wp01 = jnp.concatenate([w2bd[:, 0], w2bd[:, 1]], axis=2).astype(jnp.bfloat16)
    wp23 = jnp.concatenate([w2bd[:, 2], w2bd[:, 3]], axis=2).astype(jnp.bfloat16)
    w4 = w2bd[:, 4].astype(jnp.bfloat16)
    b2g = b2.reshape(NGRP, ROWS, 1)
    pwr = (pw / (NT // ND)).reshape(NGRP, G, ND, NH, ND)
    pwbd = jnp.einsum('gfoid,fe->gfodei', pwr, eye)
    pwbd = pwbd.reshape(NGRP, G * ND, ND * ROWS)
    pbg = pb.reshape(NGRP, G * ND, 1)

    out = pl.pallas_call(
        _kernel_body,
        grid=(NGRP,),
        in_specs=[
            pl.BlockSpec((1, G, (NT + 8) * NB), lambda g: (g, 0, 0)),
            pl.BlockSpec((1, ROWS, NK * G), lambda g: (g, 0, 0)),
            pl.BlockSpec((1, ROWS, 1), lambda g: (g, 0, 0)),
            pl.BlockSpec((1, ROWS, 2 * ROWS), lambda g: (g, 0, 0)),
            pl.BlockSpec((1, ROWS, 2 * ROWS), lambda g: (g, 0, 0)),
            pl.BlockSpec((1, ROWS, ROWS), lambda g: (g, 0, 0)),
            pl.BlockSpec((1, ROWS, 1), lambda g: (g, 0, 0)),
            pl.BlockSpec((1, G * ND, ND * ROWS), lambda g: (g, 0, 0)),
            pl.BlockSpec((1, G * ND, 1), lambda g: (g, 0, 0)),
        ],
        out_specs=pl.BlockSpec((1, G * ND, NB), lambda g: (g, 0, 0)),
        out_shape=jax.ShapeDtypeStruct((NGRP, G * ND, NB), jnp.float32),
        compiler_params=pltpu.CompilerParams(
            dimension_semantics=("parallel",),
            vmem_limit_bytes=50 * 1024 * 1024,
        ),
    )(x2, w1bd, b1g, wp01, wp23, w4, b2g, pwbd, pbg)

    return out.reshape(NF * ND, NB).T


# speedup vs baseline: 3.1142x; 1.2172x over previous
import jax
import jax.numpy as jnp
from jax.experimental import pallas as pl
from jax.experimental.pallas import tpu as pltpu

NF = 32
NB = 128
NT = 1024
NH = 16
NK = 5
ND = 4
G = 8
NGRP = NF // G
TC = 64
NC = NT // TC
E = TC + 4
CPS = (NT // ND) // TC
EB = E * NB
TB = TC * NB
ROWS = G * NH


def _dot(a, b):
    return jax.lax.dot_general(a, b, (((1,), (0,)), ((), ())),
                               preferred_element_type=jnp.float32)


def _kernel_body(x_ref, w1_ref, wp012_ref, wp34_ref, pw_ref, pb_ref, o_ref):
    xg = x_ref[0]
    w1 = w1_ref[0]
    wp012 = wp012_ref[0]
    wp34 = wp34_ref[0]
    ones8 = jnp.ones((G, EB), jnp.bfloat16)
    w3c = (E - 2) * NB
    ones1 = jnp.ones((1, w3c), jnp.bfloat16)
    segs = [None] * ND
    for c in range(NC):
        base = c * TC * NB
        xim = jnp.concatenate(
            [xg[:, base + k * NB: base + k * NB + EB] for k in range(NK)]
            + [ones8], axis=0)
        h1 = _dot(w1, xim)
        if c == 0:
            iot = jax.lax.broadcasted_iota(jnp.int32, (1, EB), 1)
            h1 = jnp.where(iot >= 2 * NB, h1, 0.0)
        if c == NC - 1:
            iot = jax.lax.broadcasted_iota(jnp.int32, (1, EB), 1)
            h1 = jnp.where(iot < (E - 2) * NB, h1, 0.0)
        h1b = jnp.maximum(h1.astype(jnp.bfloat16), 0)
        h3 = jnp.concatenate(
            [h1b[:, :w3c], h1b[:, NB: NB + w3c],
             h1b[:, 2 * NB: 2 * NB + w3c], ones1], axis=0)
        s = _dot(wp012, h3[:, :TB])
        s = s + _dot(wp34, h3[ROWS: 3 * ROWS, 2 * NB: 2 * NB + TB])
        h2 = jnp.maximum(s, 0.0)
        r = h2
        w = TB
        while w > NB:
            w //= 2
            r = r[:, :w] + r[:, w: 2 * w]
        si = c // CPS
        segs[si] = r if segs[si] is None else segs[si] + r
    pmat = jnp.concatenate(segs, axis=0)
    o_ref[0] = _dot(pw_ref[0], pmat) + pb_ref[0]


def kernel(x, lengths, w1, b1, w2, b2, pw, pb):
    del lengths
    xt = jnp.transpose(x, (2, 1, 0))
    xp = jnp.pad(xt, ((0, 0), (4, 4), (0, 0)))
    x2 = xp.reshape(NGRP, G, (NT + 8) * NB).astype(jnp.bfloat16)

    eye = jnp.eye(G, dtype=jnp.float32)
    w1r = w1[:, :, 0, :].reshape(NGRP, G, NH, NK)
    w1bd = jnp.einsum('gfck,fe->gfcke', w1r, eye).reshape(NGRP, ROWS, NK * G)
    w1bd = jnp.concatenate(
        [w1bd, b1.reshape(NGRP, ROWS, 1),
         jnp.zeros((NGRP, ROWS, G - 1), jnp.float32)], axis=2)
    w1bd = w1bd.astype(jnp.bfloat16)
    w2r = w2.reshape(NGRP, G, NH, NH, NK)
    w2bd = jnp.einsum('gfoik,fe->gkfoei', w2r, eye).reshape(NGRP, NK, ROWS, ROWS)
    wp012 = jnp.concatenate(
        [w2bd[:, 0], w2bd[:, 1], w2bd[:, 2], b2.reshape(NGRP, ROWS, 1)],
        axis=2).astype(jnp.bfloat16)
    wp34 = jnp.concatenate([w2bd[:, 3], w2bd[:, 4]], axis=2).astype(jnp.bfloat16)
    pwr = (pw / (NT // ND)).reshape(NGRP, G, ND, NH, ND)
    pwbd = jnp.einsum('gfoid,fe->gfodei', pwr, eye)
    pwbd = pwbd.reshape(NGRP, G * ND, ND * ROWS)
    pbg = pb.reshape(NGRP, G * ND, 1)

    out = pl.pallas_call(
        _kernel_body,
        grid=(NGRP,),
        in_specs=[
            pl.BlockSpec((1, G, (NT + 8) * NB), lambda g: (g, 0, 0)),
            pl.BlockSpec((1, ROWS, NK * G + G), lambda g: (g, 0, 0)),
            pl.BlockSpec((1, ROWS, 3 * ROWS + 1), lambda g: (g, 0, 0)),
            pl.BlockSpec((1, ROWS, 2 * ROWS), lambda g: (g, 0, 0)),
            pl.BlockSpec((1, G * ND, ND * ROWS), lambda g: (g, 0, 0)),
            pl.BlockSpec((1, G * ND, 1), lambda g: (g, 0, 0)),
        ],
        out_specs=pl.BlockSpec((1, G * ND, NB), lambda g: (g, 0, 0)),
        out_shape=jax.ShapeDtypeStruct((NGRP, G * ND, NB), jnp.float32),
        compiler_params=pltpu.CompilerParams(
            dimension_semantics=("parallel",),
            vmem_limit_bytes=50 * 1024 * 1024,
        ),
    )(x2, w1bd, wp012, wp34, pwbd, pbg)

    return out.reshape(NF * ND, NB).T
